# Initial kernel scaffold; baseline (speedup 1.0000x reference)
import jax
import jax.numpy as jnp
from jax.experimental import pallas as pl


def kernel(verts, edges, verts_t, edges_t, N):
    raise NotImplementedError("write your pallas kernel here")


# speedup vs baseline: 10.7054x; 10.7054x over previous
import functools

import jax
import jax.numpy as jnp
from jax import lax
from jax.experimental import pallas as pl
from jax.experimental.pallas import tpu as pltpu
from jax.experimental.pallas import tpu_sc as plsc

_NC, _NS = 2, 16
_NW = _NC * _NS
_CHUNK_EDGES = 512
_CHUNK_WORDS = 2 * _CHUNK_EDGES
_ROWS = _CHUNK_WORDS // 128


def _sqrt16(s):
    r = plsc.bitcast(jnp.int32(0x5F3759DF) - (plsc.bitcast(s, jnp.int32) >> 1),
                     jnp.float32)
    for _ in range(3):
        r = r * (jnp.float32(1.5) - jnp.float32(0.5) * s * r * r)
    return s * r


def _sqdist16(rows, ids2):
    out = None
    for c in range(3):
        col = jnp.full((16,), c, jnp.int32)
        a = plsc.load_gather(rows, [ids2, col])
        b = plsc.load_gather(rows, [ids2 + 1, col])
        d = a - b
        out = d * d if out is None else out + d * d
    return out


def _make_sc_kernel(n_chunks):
    common = n_chunks // _NW
    extra = n_chunks % _NW
    mesh = plsc.VectorSubcoreMesh(core_axis_name="c", subcore_axis_name="s")

    @functools.partial(
        pl.kernel,
        out_type=jax.ShapeDtypeStruct((_NW, 16), jnp.float32),
        mesh=mesh,
        scratch_types=[
            pltpu.VMEM((_ROWS, 128), jnp.int32),
            pltpu.VMEM((_ROWS, 128), jnp.int32),
            pltpu.VMEM((_CHUNK_WORDS, 8), jnp.float32),
            pltpu.VMEM((_CHUNK_WORDS, 8), jnp.float32),
            pltpu.VMEM((16,), jnp.float32),
            pltpu.SemaphoreType.DMA,
        ],
        compiler_params=pltpu.CompilerParams(
            needs_layout_passes=False, use_tc_tiling_on_sc=False),
    )
    def sc_kernel(eidx, etidx, verts, verts_t, out,
                  eix_v, etix_v, vrows, vtrows, acc_v, sem):
        wid = lax.axis_index("s") * _NC + lax.axis_index("c")

        def do_chunk(chunk, acc):
            row0 = chunk * _ROWS
            pltpu.sync_copy(eidx.at[pl.ds(row0, _ROWS)], eix_v)
            pltpu.sync_copy(etidx.at[pl.ds(row0, _ROWS)], etix_v)
            copies = []
            for j in range(_ROWS):
                copies.append(pltpu.async_copy(
                    verts.at[eix_v.at[j]],
                    vrows.at[pl.ds(j * 128, 128)], sem))
                copies.append(pltpu.async_copy(
                    verts_t.at[etix_v.at[j]],
                    vtrows.at[pl.ds(j * 128, 128)], sem))
            for cp in copies:
                cp.wait()

            def group(g, a):
                ids2 = (g * 16 + lax.iota(jnp.int32, 16)) * 2
                d = _sqrt16(_sqdist16(vrows, ids2))
                dt = _sqrt16(_sqdist16(vtrows, ids2))
                t = d - dt
                return a + t * t

            return lax.fori_loop(0, _CHUNK_EDGES // 16, group, acc)

        acc = lax.fori_loop(
            0, common, lambda s, a: do_chunk(s * _NW + wid, a),
            jnp.zeros((16,), jnp.float32))
        acc_v[...] = acc

        if extra:
            @pl.when(wid < extra)
            def _():
                acc_v[...] = do_chunk(common * _NW + wid, acc_v[...])

        pltpu.sync_copy(acc_v, out.at[wid])

    return sc_kernel


@functools.lru_cache(maxsize=None)
def _cached_sc_kernel(n_chunks):
    return _make_sc_kernel(n_chunks)


def _pad8(v):
    return jnp.pad(v, ((0, 0), (0, 5)))


def kernel(verts, edges, verts_t, edges_t, N):
    e_count = edges.shape[0]
    assert e_count % _CHUNK_EDGES == 0, "edge count must be chunk-aligned"
    eidx = edges.astype(jnp.int32).reshape(-1, 128)
    etidx = edges_t.astype(jnp.int32).reshape(-1, 128)
    parts = _cached_sc_kernel(e_count // _CHUNK_EDGES)(
        eidx, etidx, _pad8(verts), _pad8(verts_t))
    return (jnp.sum(parts) / N).astype(jnp.float32)


# speedup vs baseline: 10.7102x; 1.0004x over previous
import functools

import jax
import jax.numpy as jnp
from jax import lax
from jax.experimental import pallas as pl
from jax.experimental.pallas import tpu as pltpu
from jax.experimental.pallas import tpu_sc as plsc

_NC, _NS = 2, 16
_NW = _NC * _NS
_CHUNK_EDGES = 512
_CHUNK_WORDS = 2 * _CHUNK_EDGES
_ROWS = _CHUNK_WORDS // 128


def _sqrt16(s):
    r = plsc.bitcast(jnp.int32(0x5F3759DF) - (plsc.bitcast(s, jnp.int32) >> 1),
                     jnp.float32)
    for _ in range(3):
        r = r * (jnp.float32(1.5) - jnp.float32(0.5) * s * r * r)
    return s * r


def _sqdist16(rows, ids2):
    out = None
    for c in range(3):
        col = jnp.full((16,), c, jnp.int32)
        a = plsc.load_gather(rows, [ids2, col])
        b = plsc.load_gather(rows, [ids2 + 1, col])
        d = a - b
        out = d * d if out is None else out + d * d
    return out


def _make_sc_kernel(n_chunks):
    common = n_chunks // _NW
    extra = n_chunks % _NW
    mesh = plsc.VectorSubcoreMesh(core_axis_name="c", subcore_axis_name="s")

    @functools.partial(
        pl.kernel,
        out_type=jax.ShapeDtypeStruct((_NW, 16), jnp.float32),
        mesh=mesh,
        scratch_types=[
            pltpu.VMEM((_ROWS, 128), jnp.int32),
            pltpu.VMEM((_ROWS, 128), jnp.int32),
            pltpu.VMEM((_CHUNK_WORDS, 8), jnp.float32),
            pltpu.VMEM((_CHUNK_WORDS, 8), jnp.float32),
            pltpu.VMEM((16,), jnp.float32),
            pltpu.SemaphoreType.DMA,
        ],
        compiler_params=pltpu.CompilerParams(
            needs_layout_passes=False, use_tc_tiling_on_sc=False),
    )
    def sc_kernel(eidx, etidx, verts, verts_t, out,
                  eix_v, etix_v, vrows, vtrows, acc_v, sem):
        wid = lax.axis_index("s") * _NC + lax.axis_index("c")

        def do_chunk(chunk, acc):
            pltpu.sync_copy(eidx.at[chunk], eix_v)
            pltpu.sync_copy(etidx.at[chunk], etix_v)
            copies = []
            for j in range(_ROWS):
                copies.append(pltpu.async_copy(
                    verts.at[eix_v.at[j]],
                    vrows.at[pl.ds(j * 128, 128)], sem))
                copies.append(pltpu.async_copy(
                    verts_t.at[etix_v.at[j]],
                    vtrows.at[pl.ds(j * 128, 128)], sem))
            for cp in copies:
                cp.wait()

            def group(g, a):
                ids2 = (g * 16 + lax.iota(jnp.int32, 16)) * 2
                d = _sqrt16(_sqdist16(vrows, ids2))
                dt = _sqrt16(_sqdist16(vtrows, ids2))
                t = d - dt
                return a + t * t

            return lax.fori_loop(0, _CHUNK_EDGES // 16, group, acc)

        acc = lax.fori_loop(
            0, common, lambda s, a: do_chunk(s * _NW + wid, a),
            jnp.zeros((16,), jnp.float32))
        acc_v[...] = acc

        if extra:
            @pl.when(wid < extra)
            def _():
                acc_v[...] = do_chunk(common * _NW + wid, acc_v[...])

        pltpu.sync_copy(acc_v, out.at[wid])

    return sc_kernel


@functools.lru_cache(maxsize=None)
def _cached_sc_kernel(n_chunks):
    return _make_sc_kernel(n_chunks)


def _pad8(v):
    return jnp.pad(v, ((0, 0), (0, 5)))


def kernel(verts, edges, verts_t, edges_t, N):
    e_count = edges.shape[0]
    assert e_count % _CHUNK_EDGES == 0, "edge count must be chunk-aligned"
    eidx = edges.astype(jnp.int32).reshape(-1, _ROWS, 128)
    etidx = edges_t.astype(jnp.int32).reshape(-1, _ROWS, 128)
    parts = _cached_sc_kernel(e_count // _CHUNK_EDGES)(
        eidx, etidx, _pad8(verts), _pad8(verts_t))
    return (jnp.sum(parts) / N).astype(jnp.float32)


# speedup vs baseline: 67.3190x; 6.2855x over previous
import functools

import jax
import jax.numpy as jnp
from jax import lax
from jax.experimental import pallas as pl
from jax.experimental.pallas import tpu as pltpu
from jax.experimental.pallas import tpu_sc as plsc

_NC, _NS = 2, 16
_NW = _NC * _NS
_CHUNK_EDGES = 512
_CHUNK_WORDS = 2 * _CHUNK_EDGES
_BLOCKS_PER_CHUNK = _CHUNK_EDGES // 128


def _sqrt16(s):
    r = plsc.bitcast(jnp.int32(0x5F3759DF) - (plsc.bitcast(s, jnp.int32) >> 1),
                     jnp.float32)
    for _ in range(3):
        r = r * (jnp.float32(1.5) - jnp.float32(0.5) * s * r * r)
    return s * r


def _sqdist16(rows, ids0):
    out = None
    for c in range(3):
        col = jnp.full((16,), c, jnp.int32)
        a = plsc.load_gather(rows, [ids0, col])
        b = plsc.load_gather(rows, [ids0 + 128, col])
        d = a - b
        out = d * d if out is None else out + d * d
    return out


def _make_sc_kernel(n_chunks):
    common = n_chunks // _NW
    extra = n_chunks % _NW
    mesh = plsc.VectorSubcoreMesh(core_axis_name="c", subcore_axis_name="s")

    @functools.partial(
        pl.kernel,
        out_type=jax.ShapeDtypeStruct((_NW, 16), jnp.float32),
        mesh=mesh,
        scratch_types=[
            pltpu.VMEM((_BLOCKS_PER_CHUNK, 2, 128), jnp.int32),
            pltpu.VMEM((_BLOCKS_PER_CHUNK, 2, 128), jnp.int32),
            pltpu.VMEM((_CHUNK_WORDS, 8), jnp.float32),
            pltpu.VMEM((_CHUNK_WORDS, 8), jnp.float32),
            pltpu.VMEM((16,), jnp.float32),
            pltpu.SemaphoreType.DMA,
        ],
        compiler_params=pltpu.CompilerParams(
            needs_layout_passes=False, use_tc_tiling_on_sc=False),
    )
    def sc_kernel(eidx, etidx, verts, verts_t, out,
                  eix_v, etix_v, vrows, vtrows, acc_v, sem):
        wid = lax.axis_index("s") * _NC + lax.axis_index("c")

        def do_chunk(chunk, acc):
            blk0 = chunk * _BLOCKS_PER_CHUNK
            pltpu.sync_copy(eidx.at[pl.ds(blk0, _BLOCKS_PER_CHUNK)], eix_v)
            pltpu.sync_copy(etidx.at[pl.ds(blk0, _BLOCKS_PER_CHUNK)], etix_v)
            copies = []
            for b in range(_BLOCKS_PER_CHUNK):
                for j in range(2):
                    copies.append(pltpu.async_copy(
                        verts.at[eix_v.at[b, j]],
                        vrows.at[pl.ds((2 * b + j) * 128, 128)], sem))
                    copies.append(pltpu.async_copy(
                        verts_t.at[etix_v.at[b, j]],
                        vtrows.at[pl.ds((2 * b + j) * 128, 128)], sem))
            for cp in copies:
                cp.wait()

            def group(g, a):
                ids0 = (g // 8) * 256 + (g % 8) * 16 + lax.iota(jnp.int32, 16)
                d = _sqrt16(_sqdist16(vrows, ids0))
                dt = _sqrt16(_sqdist16(vtrows, ids0))
                t = d - dt
                return a + t * t

            return lax.fori_loop(0, _CHUNK_EDGES // 16, group, acc)

        acc = lax.fori_loop(
            0, common, lambda s, a: do_chunk(s * _NW + wid, a),
            jnp.zeros((16,), jnp.float32))
        acc_v[...] = acc

        if extra:
            @pl.when(wid < extra)
            def _():
                acc_v[...] = do_chunk(common * _NW + wid, acc_v[...])

        pltpu.sync_copy(acc_v, out.at[wid])

    return sc_kernel


@functools.lru_cache(maxsize=None)
def _cached_sc_kernel(n_chunks):
    return _make_sc_kernel(n_chunks)


def _pad8(v):
    return jnp.pad(v, ((0, 0), (0, 5)))


def kernel(verts, edges, verts_t, edges_t, N):
    e_count = edges.shape[0]
    assert e_count % _CHUNK_EDGES == 0, "edge count must be chunk-aligned"
    eidx = edges.astype(jnp.int32).reshape(-1, 128, 2).transpose(0, 2, 1)
    etidx = edges_t.astype(jnp.int32).reshape(-1, 128, 2).transpose(0, 2, 1)
    parts = _cached_sc_kernel(e_count // _CHUNK_EDGES)(
        eidx, etidx, _pad8(verts), _pad8(verts_t))
    return (jnp.sum(parts) / N).astype(jnp.float32)


# speedup vs baseline: 103.8965x; 1.5433x over previous
import functools

import jax
import jax.numpy as jnp
from jax import lax
from jax.experimental import pallas as pl
from jax.experimental.pallas import tpu as pltpu
from jax.experimental.pallas import tpu_sc as plsc

_NC, _NS = 2, 16
_NW = _NC * _NS
_CHUNK_EDGES = 512
_CHUNK_WORDS = 2 * _CHUNK_EDGES
_BLOCKS_PER_CHUNK = _CHUNK_EDGES // 128


def _sqrt16(s):
    r = plsc.bitcast(jnp.int32(0x5F3759DF) - (plsc.bitcast(s, jnp.int32) >> 1),
                     jnp.float32)
    for _ in range(3):
        r = r * (jnp.float32(1.5) - jnp.float32(0.5) * s * r * r)
    return s * r


def _sqdist16(rows, ids0):
    out = None
    for c in range(3):
        col = jnp.full((16,), c, jnp.int32)
        a = plsc.load_gather(rows, [ids0, col])
        b = plsc.load_gather(rows, [ids0 + 128, col])
        d = a - b
        out = d * d if out is None else out + d * d
    return out


def _make_sc_kernel(n_chunks):
    base = n_chunks // _NW
    rem = n_chunks % _NW
    _BPC = _BLOCKS_PER_CHUNK
    mesh = plsc.VectorSubcoreMesh(core_axis_name="c", subcore_axis_name="s")

    @functools.partial(
        pl.kernel,
        out_type=jax.ShapeDtypeStruct((_NW, 16), jnp.float32),
        mesh=mesh,
        scratch_types=[
            pltpu.VMEM((2, _BPC, 2, 128), jnp.int32),
            pltpu.VMEM((2, _BPC, 2, 128), jnp.int32),
            pltpu.VMEM((2, _CHUNK_WORDS, 8), jnp.float32),
            pltpu.VMEM((2, _CHUNK_WORDS, 8), jnp.float32),
            pltpu.VMEM((16,), jnp.float32),
            pltpu.SemaphoreType.DMA,
            pltpu.SemaphoreType.DMA,
            pltpu.SemaphoreType.DMA,
            pltpu.SemaphoreType.DMA,
        ],
        compiler_params=pltpu.CompilerParams(
            needs_layout_passes=False, use_tc_tiling_on_sc=False),
    )
    def sc_kernel(eidx, etidx, verts, verts_t, out,
                  eix_v, etix_v, vrows, vtrows, acc_v,
                  isem0, isem1, gsem0, gsem1):
        wid = lax.axis_index("s") * _NC + lax.axis_index("c")
        cnt = jnp.where(wid < rem, base + 1, base)
        start = wid * base + jnp.minimum(wid, rem)
        isems = (isem0, isem1)
        gsems = (gsem0, gsem1)

        def idx_start(c, p):
            blk0 = c * _BPC
            pltpu.async_copy(eidx.at[pl.ds(blk0, _BPC)], eix_v.at[p],
                             isems[p])
            pltpu.async_copy(etidx.at[pl.ds(blk0, _BPC)], etix_v.at[p],
                             isems[p])

        def idx_wait(p):
            pltpu.make_async_copy(eidx.at[pl.ds(0, _BPC)], eix_v.at[p],
                                  isems[p]).wait()
            pltpu.make_async_copy(etidx.at[pl.ds(0, _BPC)], etix_v.at[p],
                                  isems[p]).wait()

        def gather_start(p):
            for b in range(_BPC):
                for j in range(2):
                    row0 = (2 * b + j) * 128
                    pltpu.async_copy(
                        verts.at[eix_v.at[p, b, j]],
                        vrows.at[p].at[pl.ds(row0, 128)], gsems[p])
                    pltpu.async_copy(
                        verts_t.at[etix_v.at[p, b, j]],
                        vtrows.at[p].at[pl.ds(row0, 128)], gsems[p])

        def gather_wait(p):
            pltpu.make_async_copy(verts.at[pl.ds(0, _CHUNK_WORDS)],
                                  vrows.at[p], gsems[p]).wait()
            pltpu.make_async_copy(verts_t.at[pl.ds(0, _CHUNK_WORDS)],
                                  vtrows.at[p], gsems[p]).wait()

        def compute(p):
            def group(g, a):
                ids0 = ((g // 8) * 256 + (g % 8) * 16
                        + lax.iota(jnp.int32, 16))
                d = _sqrt16(_sqdist16(vrows.at[p], ids0))
                dt = _sqrt16(_sqdist16(vtrows.at[p], ids0))
                t = d - dt
                return a + t * t

            acc_v[...] = lax.fori_loop(0, _CHUNK_EDGES // 16, group,
                                       acc_v[...])

        def step(k, p, q):
            idx_wait(p)
            gather_start(p)
            gather_wait(q)

            @pl.when(k + 1 < cnt)
            def _():
                idx_start(start + k + 1, q)

            compute(q)

        acc_v[...] = jnp.zeros((16,), jnp.float32)

        idx_start(start, 0)
        idx_wait(0)
        gather_start(0)

        @pl.when(cnt > 1)
        def _():
            idx_start(start + 1, 1)

        def pair_body(s, carry):
            step(2 * s + 1, 1, 0)
            step(2 * s + 2, 0, 1)
            return carry

        lax.fori_loop(0, (cnt - 1) // 2, pair_body, 0)

        @pl.when((cnt - 1) % 2 == 1)
        def _():
            step(cnt - 1, 1, 0)

        @pl.when((cnt - 1) % 2 == 0)
        def _():
            gather_wait(0)
            compute(0)

        @pl.when((cnt - 1) % 2 == 1)
        def _():
            gather_wait(1)
            compute(1)

        pltpu.sync_copy(acc_v, out.at[wid])

    return sc_kernel


@functools.lru_cache(maxsize=None)
def _cached_sc_kernel(n_chunks):
    return _make_sc_kernel(n_chunks)


def _pad8(v):
    return jnp.pad(v, ((0, 0), (0, 5)))


def kernel(verts, edges, verts_t, edges_t, N):
    e_count = edges.shape[0]
    assert e_count % _CHUNK_EDGES == 0, "edge count must be chunk-aligned"
    eidx = edges.astype(jnp.int32).reshape(-1, 128, 2).transpose(0, 2, 1)
    etidx = edges_t.astype(jnp.int32).reshape(-1, 128, 2).transpose(0, 2, 1)
    parts = _cached_sc_kernel(e_count // _CHUNK_EDGES)(
        eidx, etidx, _pad8(verts), _pad8(verts_t))
    return (jnp.sum(parts) / N).astype(jnp.float32)


# speedup vs baseline: 289.1873x; 2.7834x over previous
import functools

import jax
import jax.numpy as jnp
from jax import lax
from jax.experimental import pallas as pl
from jax.experimental.pallas import tpu as pltpu
from jax.experimental.pallas import tpu_sc as plsc

_NC, _NS = 2, 16
_NW = _NC * _NS
_CHUNK_EDGES = 512
_CHUNK_WORDS = 2 * _CHUNK_EDGES
_BLOCKS_PER_CHUNK = _CHUNK_EDGES // 128


def _sqrt16(s):
    r = plsc.bitcast(jnp.int32(0x5F3759DF) - (plsc.bitcast(s, jnp.int32) >> 1),
                     jnp.float32)
    for _ in range(3):
        r = r * (jnp.float32(1.5) - jnp.float32(0.5) * s * r * r)
    return s * r


def _sqdist16(rows, ids0, col0):
    out = None
    for c in range(col0, col0 + 3):
        col = jnp.full((16,), c, jnp.int32)
        a = plsc.load_gather(rows, [ids0, col])
        b = plsc.load_gather(rows, [ids0 + 128, col])
        d = a - b
        out = d * d if out is None else out + d * d
    return out


def _make_sc_kernel(n_chunks, n_verts):
    base = n_chunks // _NW
    rem = n_chunks % _NW
    _BPC = _BLOCKS_PER_CHUNK
    mesh = plsc.VectorSubcoreMesh(core_axis_name="c", subcore_axis_name="s")

    @functools.partial(
        pl.kernel,
        out_type=jax.ShapeDtypeStruct((_NW, 16), jnp.float32),
        mesh=mesh,
        scratch_types=[
            pltpu.VMEM((2, _BPC, 2, 128), jnp.int32),
            pltpu.VMEM((2, _BPC, 2, 128), jnp.int32),
            pltpu.VMEM((2, _CHUNK_WORDS, 8), jnp.float32),
            pltpu.VMEM((2, _CHUNK_WORDS, 8), jnp.float32),
            pltpu.VMEM((16,), jnp.float32),
            pltpu.VMEM_SHARED((n_verts, 8), jnp.float32),
            pltpu.SemaphoreType.DMA,
            pltpu.SemaphoreType.DMA,
            pltpu.SemaphoreType.DMA,
            pltpu.SemaphoreType.DMA,
        ],
        compiler_params=pltpu.CompilerParams(
            needs_layout_passes=False, use_tc_tiling_on_sc=False),
    )
    def sc_kernel(eidx, etidx, table, out,
                  eix_v, etix_v, vrows, vtrows, acc_v, stab,
                  isem0, isem1, gsem0, gsem1):
        wid = lax.axis_index("s") * _NC + lax.axis_index("c")

        @pl.when(lax.axis_index("s") == 0)
        def _():
            pltpu.sync_copy(table, stab)

        plsc.subcore_barrier()
        cnt = jnp.where(wid < rem, base + 1, base)
        start = wid * base + jnp.minimum(wid, rem)
        isems = (isem0, isem1)
        gsems = (gsem0, gsem1)

        def idx_start(c, p):
            blk0 = c * _BPC
            pltpu.async_copy(eidx.at[pl.ds(blk0, _BPC)], eix_v.at[p],
                             isems[p])
            pltpu.async_copy(etidx.at[pl.ds(blk0, _BPC)], etix_v.at[p],
                             isems[p])

        def idx_wait(p):
            pltpu.make_async_copy(eidx.at[pl.ds(0, _BPC)], eix_v.at[p],
                                  isems[p]).wait()
            pltpu.make_async_copy(etidx.at[pl.ds(0, _BPC)], etix_v.at[p],
                                  isems[p]).wait()

        def gather_start(p):
            for b in range(_BPC):
                for j in range(2):
                    row0 = (2 * b + j) * 128
                    pltpu.async_copy(
                        stab.at[eix_v.at[p, b, j]],
                        vrows.at[p].at[pl.ds(row0, 128)], gsems[p])
                    pltpu.async_copy(
                        stab.at[etix_v.at[p, b, j]],
                        vtrows.at[p].at[pl.ds(row0, 128)], gsems[p])

        def gather_wait(p):
            pltpu.make_async_copy(table.at[pl.ds(0, _CHUNK_WORDS)],
                                  vrows.at[p], gsems[p]).wait()
            pltpu.make_async_copy(table.at[pl.ds(0, _CHUNK_WORDS)],
                                  vtrows.at[p], gsems[p]).wait()

        def compute(p):
            def group(g, a):
                ids0 = ((g // 8) * 256 + (g % 8) * 16
                        + lax.iota(jnp.int32, 16))
                d = _sqrt16(_sqdist16(vrows.at[p], ids0, 0))
                dt = _sqrt16(_sqdist16(vtrows.at[p], ids0, 3))
                t = d - dt
                return a + t * t

            acc_v[...] = lax.fori_loop(0, _CHUNK_EDGES // 16, group,
                                       acc_v[...])

        def step(k, p, q):
            idx_wait(p)
            gather_start(p)
            gather_wait(q)

            @pl.when(k + 1 < cnt)
            def _():
                idx_start(start + k + 1, q)

            compute(q)

        acc_v[...] = jnp.zeros((16,), jnp.float32)

        idx_start(start, 0)
        idx_wait(0)
        gather_start(0)

        @pl.when(cnt > 1)
        def _():
            idx_start(start + 1, 1)

        def pair_body(s, carry):
            step(2 * s + 1, 1, 0)
            step(2 * s + 2, 0, 1)
            return carry

        lax.fori_loop(0, (cnt - 1) // 2, pair_body, 0)

        @pl.when((cnt - 1) % 2 == 1)
        def _():
            step(cnt - 1, 1, 0)

        @pl.when((cnt - 1) % 2 == 0)
        def _():
            gather_wait(0)
            compute(0)

        @pl.when((cnt - 1) % 2 == 1)
        def _():
            gather_wait(1)
            compute(1)

        pltpu.sync_copy(acc_v, out.at[wid])

    return sc_kernel


@functools.lru_cache(maxsize=None)
def _cached_sc_kernel(n_chunks, n_verts):
    return _make_sc_kernel(n_chunks, n_verts)


def kernel(verts, edges, verts_t, edges_t, N):
    e_count = edges.shape[0]
    assert e_count % _CHUNK_EDGES == 0, "edge count must be chunk-aligned"
    eidx = edges.astype(jnp.int32).reshape(-1, 128, 2).transpose(0, 2, 1)
    etidx = edges_t.astype(jnp.int32).reshape(-1, 128, 2).transpose(0, 2, 1)
    table = jnp.concatenate(
        [verts, verts_t, jnp.zeros((verts.shape[0], 2), jnp.float32)],
        axis=1)
    parts = _cached_sc_kernel(e_count // _CHUNK_EDGES, verts.shape[0])(
        eidx, etidx, table)
    return (jnp.sum(parts) / N).astype(jnp.float32)


# speedup vs baseline: 410.5419x; 1.4196x over previous
import functools

import jax
import jax.numpy as jnp
from jax import lax
from jax.experimental import pallas as pl
from jax.experimental.pallas import tpu as pltpu
from jax.experimental.pallas import tpu_sc as plsc

_NC, _NS = 2, 16
_NW = _NC * _NS
_CHUNK_EDGES = 512
_CHUNK_WORDS = 2 * _CHUNK_EDGES
_BLOCKS_PER_CHUNK = _CHUNK_EDGES // 128


def _sqrt16(s):
    r = plsc.bitcast(jnp.int32(0x5F3759DF) - (plsc.bitcast(s, jnp.int32) >> 1),
                     jnp.float32)
    for _ in range(3):
        r = r * (jnp.float32(1.5) - jnp.float32(0.5) * s * r * r)
    return s * r


_BLOCK_V = 128


def _sqdist16(rows, ids0, col0):
    out = None
    for c in range(col0, col0 + 3):
        col = jnp.full((16,), c, jnp.int32)
        a = plsc.load_gather(rows, [ids0, col])
        b = plsc.load_gather(rows, [ids0 + 128, col])
        d = a - b
        out = d * d if out is None else out + d * d
    return out


def _make_sc_kernel(n_chunks, n_vp):
    base = n_chunks // _NW
    rem = n_chunks % _NW
    _BPC = _BLOCKS_PER_CHUNK
    n_vblocks = n_vp // _BLOCK_V
    vb_per_sub = n_vblocks // _NS
    mesh = plsc.VectorSubcoreMesh(core_axis_name="c", subcore_axis_name="s")

    @functools.partial(
        pl.kernel,
        out_type=jax.ShapeDtypeStruct((_NW, 16), jnp.float32),
        mesh=mesh,
        scratch_types=[
            pltpu.VMEM((2, _BPC, 2, 128), jnp.int32),
            pltpu.VMEM((2, _BPC, 2, 128), jnp.int32),
            pltpu.VMEM((2, _CHUNK_WORDS, 8), jnp.float32),
            pltpu.VMEM((2, _CHUNK_WORDS, 8), jnp.float32),
            pltpu.VMEM((16,), jnp.float32),
            pltpu.VMEM_SHARED((n_vp, 8), jnp.float32),
            pltpu.VMEM((((vb_per_sub + 1) // 2) * 1024,), jnp.float32),
            pltpu.VMEM((2, _BLOCK_V, 8), jnp.float32),
            pltpu.SemaphoreType.DMA,
            pltpu.SemaphoreType.DMA,
            pltpu.SemaphoreType.DMA,
            pltpu.SemaphoreType.DMA,
            pltpu.SemaphoreType.DMA,
        ],
        compiler_params=pltpu.CompilerParams(
            needs_layout_passes=False, use_tc_tiling_on_sc=False),
    )
    def sc_kernel(eidx, etidx, tview, dummy, out,
                  eix_v, etix_v, vrows, vtrows, acc_v, stab,
                  inslab, tbuf, isem0, isem1, gsem0, gsem1, bsem):
        wid = lax.axis_index("s") * _NC + lax.axis_index("c")
        sid = lax.axis_index("s")

        vb0 = sid * vb_per_sub
        iota16 = lax.iota(jnp.int32, 16)

        def phase0_round(nblk, off):
            pltpu.sync_copy(
                tview.at[pl.ds((vb0 + off) * 1024, nblk * 1024)],
                inslab.at[pl.ds(0, nblk * 1024)])

            def build_block(l, p):
                @pl.when(l >= 2)
                def _():
                    pltpu.make_async_copy(tbuf.at[p],
                                          stab.at[pl.ds(0, _BLOCK_V)],
                                          bsem).wait()
                base = l * 1024
                for q in range(_BLOCK_V // 16):
                    r_ids = q * 16 + iota16
                    for c in range(6):
                        v = plsc.load_gather(
                            inslab, [base + c * 128 + r_ids])
                        plsc.store_scatter(
                            tbuf.at[p],
                            [r_ids, jnp.full((16,), c, jnp.int32)], v)
                pltpu.async_copy(
                    tbuf.at[p],
                    stab.at[pl.ds((vb0 + off + l) * _BLOCK_V, _BLOCK_V)],
                    bsem)

            def build_pair(i, carry):
                build_block(2 * i, 0)
                build_block(2 * i + 1, 1)
                return carry

            lax.fori_loop(0, nblk // 2, build_pair, 0)
            if nblk % 2:
                build_block(jnp.int32(nblk - 1), 0)
            pltpu.make_async_copy(tbuf.at[0], stab.at[pl.ds(0, _BLOCK_V)],
                                  bsem).wait()
            pltpu.make_async_copy(tbuf.at[1], stab.at[pl.ds(0, _BLOCK_V)],
                                  bsem).wait()

        h1 = (vb_per_sub + 1) // 2
        phase0_round(h1, 0)
        phase0_round(vb_per_sub - h1, h1)
        plsc.subcore_barrier()
        cnt = jnp.where(wid < rem, base + 1, base)
        start = wid * base + jnp.minimum(wid, rem)
        isems = (isem0, isem1)
        gsems = (gsem0, gsem1)

        def idx_start(c, p):
            blk0 = c * _BPC
            pltpu.async_copy(eidx.at[pl.ds(blk0, _BPC)], eix_v.at[p],
                             isems[p])
            pltpu.async_copy(etidx.at[pl.ds(blk0, _BPC)], etix_v.at[p],
                             isems[p])

        def idx_wait(p):
            pltpu.make_async_copy(eidx.at[pl.ds(0, _BPC)], eix_v.at[p],
                                  isems[p]).wait()
            pltpu.make_async_copy(etidx.at[pl.ds(0, _BPC)], etix_v.at[p],
                                  isems[p]).wait()

        def gather_start(p):
            for b in range(_BPC):
                for j in range(2):
                    row0 = (2 * b + j) * 128
                    pltpu.async_copy(
                        stab.at[eix_v.at[p, b, j]],
                        vrows.at[p].at[pl.ds(row0, 128)], gsems[p])
                    pltpu.async_copy(
                        stab.at[etix_v.at[p, b, j]],
                        vtrows.at[p].at[pl.ds(row0, 128)], gsems[p])

        def gather_wait(p):
            pltpu.make_async_copy(dummy, vrows.at[p], gsems[p]).wait()
            pltpu.make_async_copy(dummy, vtrows.at[p], gsems[p]).wait()

        def compute(p):
            def group(g, a):
                ids0 = ((g // 8) * 256 + (g % 8) * 16
                        + lax.iota(jnp.int32, 16))
                d = _sqrt16(_sqdist16(vrows.at[p], ids0, 0))
                dt = _sqrt16(_sqdist16(vtrows.at[p], ids0, 3))
                t = d - dt
                return a + t * t

            acc_v[...] = lax.fori_loop(0, _CHUNK_EDGES // 16, group,
                                       acc_v[...])

        def step(k, p, q):
            idx_wait(p)
            gather_start(p)
            gather_wait(q)

            @pl.when(k + 1 < cnt)
            def _():
                idx_start(start + k + 1, q)

            compute(q)

        acc_v[...] = jnp.zeros((16,), jnp.float32)

        idx_start(start, 0)
        idx_wait(0)
        gather_start(0)

        @pl.when(cnt > 1)
        def _():
            idx_start(start + 1, 1)

        def pair_body(s, carry):
            step(2 * s + 1, 1, 0)
            step(2 * s + 2, 0, 1)
            return carry

        lax.fori_loop(0, (cnt - 1) // 2, pair_body, 0)

        @pl.when((cnt - 1) % 2 == 1)
        def _():
            step(cnt - 1, 1, 0)

        @pl.when((cnt - 1) % 2 == 0)
        def _():
            gather_wait(0)
            compute(0)

        @pl.when((cnt - 1) % 2 == 1)
        def _():
            gather_wait(1)
            compute(1)

        pltpu.sync_copy(acc_v, out.at[wid])

    return sc_kernel


@functools.lru_cache(maxsize=None)
def _cached_sc_kernel(n_chunks, n_verts):
    return _make_sc_kernel(n_chunks, n_verts)


def kernel(verts, edges, verts_t, edges_t, N):
    e_count = edges.shape[0]
    assert e_count % _CHUNK_EDGES == 0, "edge count must be chunk-aligned"
    eidx = edges.astype(jnp.int32).reshape(-1, 128, 2).transpose(0, 2, 1)
    etidx = edges_t.astype(jnp.int32).reshape(-1, 128, 2).transpose(0, 2, 1)
    v_count = verts.shape[0]
    vp = -(-v_count // (_NS * _BLOCK_V)) * (_NS * _BLOCK_V)
    pad = ((0, vp - v_count), (0, 0))
    table = jnp.concatenate(
        [jnp.pad(verts, pad), jnp.pad(verts_t, pad),
         jnp.zeros((vp, 2), jnp.float32)], axis=1)
    tview = table.reshape(-1, 128, 8).transpose(0, 2, 1).reshape(-1)
    dummy = jnp.zeros((_CHUNK_WORDS, 8), jnp.float32)
    parts = _cached_sc_kernel(e_count // _CHUNK_EDGES, vp)(
        eidx, etidx, tview, dummy)
    return (jnp.sum(parts) / N).astype(jnp.float32)


# speedup vs baseline: 411.3130x; 1.0019x over previous
import functools

import jax
import jax.numpy as jnp
from jax import lax
from jax.experimental import pallas as pl
from jax.experimental.pallas import tpu as pltpu
from jax.experimental.pallas import tpu_sc as plsc

_NC, _NS = 2, 16
_NW = _NC * _NS
_CHUNK_EDGES = 512
_CHUNK_WORDS = 2 * _CHUNK_EDGES
_BLOCKS_PER_CHUNK = _CHUNK_EDGES // 128


def _sqrt16(s):
    r = plsc.bitcast(jnp.int32(0x5F3759DF) - (plsc.bitcast(s, jnp.int32) >> 1),
                     jnp.float32)
    for _ in range(3):
        r = r * (jnp.float32(1.5) - jnp.float32(0.5) * s * r * r)
    return s * r


_BLOCK_V = 128


def _sqdist16(rows, ids0, col0):
    out = None
    for c in range(col0, col0 + 3):
        col = jnp.full((16,), c, jnp.int32)
        a = plsc.load_gather(rows, [ids0, col])
        b = plsc.load_gather(rows, [ids0 + 128, col])
        d = a - b
        out = d * d if out is None else out + d * d
    return out


def _make_sc_kernel(n_chunks, n_vp):
    base = n_chunks // _NW
    rem = n_chunks % _NW
    assert base >= 2, "each subcore needs at least two chunks"
    _BPC = _BLOCKS_PER_CHUNK
    n_vblocks = n_vp // _BLOCK_V
    vb_per_sub = n_vblocks // _NS
    mesh = plsc.VectorSubcoreMesh(core_axis_name="c", subcore_axis_name="s")

    @functools.partial(
        pl.kernel,
        out_type=jax.ShapeDtypeStruct((_NW, 16), jnp.float32),
        mesh=mesh,
        scratch_types=[
            pltpu.VMEM((2, _BPC, 2, 128), jnp.int32),
            pltpu.VMEM((2, _BPC, 2, 128), jnp.int32),
            pltpu.VMEM((2, _CHUNK_WORDS, 8), jnp.float32),
            pltpu.VMEM((2, _CHUNK_WORDS, 8), jnp.float32),
            pltpu.VMEM((16,), jnp.float32),
            pltpu.VMEM_SHARED((n_vp, 8), jnp.float32),
            pltpu.VMEM((((vb_per_sub + 1) // 2) * 1024,), jnp.float32),
            pltpu.VMEM((2, _BLOCK_V, 8), jnp.float32),
            pltpu.SemaphoreType.DMA,
            pltpu.SemaphoreType.DMA,
            pltpu.SemaphoreType.DMA,
            pltpu.SemaphoreType.DMA,
            pltpu.SemaphoreType.DMA,
        ],
        compiler_params=pltpu.CompilerParams(
            needs_layout_passes=False, use_tc_tiling_on_sc=False),
    )
    def sc_kernel(eidx, etidx, tview, dummy, out,
                  eix_v, etix_v, vrows, vtrows, acc_v, stab,
                  inslab, tbuf, isem0, isem1, gsem0, gsem1, bsem):
        wid = lax.axis_index("s") * _NC + lax.axis_index("c")
        sid = lax.axis_index("s")

        vb0 = sid * vb_per_sub
        iota16 = lax.iota(jnp.int32, 16)
        cnt = jnp.where(wid < rem, base + 1, base)
        start = wid * base + jnp.minimum(wid, rem)
        isems = (isem0, isem1)
        gsems = (gsem0, gsem1)

        def idx_start(c, p):
            blk0 = c * _BPC
            pltpu.async_copy(eidx.at[pl.ds(blk0, _BPC)], eix_v.at[p],
                             isems[p])
            pltpu.async_copy(etidx.at[pl.ds(blk0, _BPC)], etix_v.at[p],
                             isems[p])

        idx_start(start, 0)
        idx_start(start + 1, 1)

        def phase0_round(nblk, off):
            pltpu.sync_copy(
                tview.at[pl.ds((vb0 + off) * 1024, nblk * 1024)],
                inslab.at[pl.ds(0, nblk * 1024)])

            def build_block(l, p):
                @pl.when(l >= 2)
                def _():
                    pltpu.make_async_copy(tbuf.at[p],
                                          stab.at[pl.ds(0, _BLOCK_V)],
                                          bsem).wait()
                base = l * 1024
                for q in range(_BLOCK_V // 16):
                    r_ids = q * 16 + iota16
                    for c in range(6):
                        v = plsc.load_gather(
                            inslab, [base + c * 128 + r_ids])
                        plsc.store_scatter(
                            tbuf.at[p],
                            [r_ids, jnp.full((16,), c, jnp.int32)], v)
                pltpu.async_copy(
                    tbuf.at[p],
                    stab.at[pl.ds((vb0 + off + l) * _BLOCK_V, _BLOCK_V)],
                    bsem)

            def build_pair(i, carry):
                build_block(2 * i, 0)
                build_block(2 * i + 1, 1)
                return carry

            lax.fori_loop(0, nblk // 2, build_pair, 0)
            if nblk % 2:
                build_block(jnp.int32(nblk - 1), 0)
            pltpu.make_async_copy(tbuf.at[0], stab.at[pl.ds(0, _BLOCK_V)],
                                  bsem).wait()
            pltpu.make_async_copy(tbuf.at[1], stab.at[pl.ds(0, _BLOCK_V)],
                                  bsem).wait()

        h1 = (vb_per_sub + 1) // 2
        phase0_round(h1, 0)
        phase0_round(vb_per_sub - h1, h1)
        plsc.subcore_barrier()

        def idx_wait(p):
            pltpu.make_async_copy(eidx.at[pl.ds(0, _BPC)], eix_v.at[p],
                                  isems[p]).wait()
            pltpu.make_async_copy(etidx.at[pl.ds(0, _BPC)], etix_v.at[p],
                                  isems[p]).wait()

        def gather_start(p):
            for b in range(_BPC):
                for j in range(2):
                    row0 = (2 * b + j) * 128
                    pltpu.async_copy(
                        stab.at[eix_v.at[p, b, j]],
                        vrows.at[p].at[pl.ds(row0, 128)], gsems[p])
                    pltpu.async_copy(
                        stab.at[etix_v.at[p, b, j]],
                        vtrows.at[p].at[pl.ds(row0, 128)], gsems[p])

        def gather_wait(p):
            pltpu.make_async_copy(dummy, vrows.at[p], gsems[p]).wait()
            pltpu.make_async_copy(dummy, vtrows.at[p], gsems[p]).wait()

        def compute(p):
            def group(g, a):
                ids0 = ((g // 8) * 256 + (g % 8) * 16
                        + lax.iota(jnp.int32, 16))
                d = _sqrt16(_sqdist16(vrows.at[p], ids0, 0))
                dt = _sqrt16(_sqdist16(vtrows.at[p], ids0, 3))
                t = d - dt
                return a + t * t

            acc_v[...] = lax.fori_loop(0, _CHUNK_EDGES // 16, group,
                                       acc_v[...])

        def step(k, p, q):
            idx_wait(p)
            gather_start(p)
            gather_wait(q)

            @pl.when(k + 1 < cnt)
            def _():
                idx_start(start + k + 1, q)

            compute(q)

        acc_v[...] = jnp.zeros((16,), jnp.float32)

        idx_wait(0)
        gather_start(0)

        def pair_body(s, carry):
            step(2 * s + 1, 1, 0)
            step(2 * s + 2, 0, 1)
            return carry

        lax.fori_loop(0, (cnt - 1) // 2, pair_body, 0)

        @pl.when((cnt - 1) % 2 == 1)
        def _():
            step(cnt - 1, 1, 0)

        @pl.when((cnt - 1) % 2 == 0)
        def _():
            gather_wait(0)
            compute(0)

        @pl.when((cnt - 1) % 2 == 1)
        def _():
            gather_wait(1)
            compute(1)

        pltpu.sync_copy(acc_v, out.at[wid])

    return sc_kernel


@functools.lru_cache(maxsize=None)
def _cached_sc_kernel(n_chunks, n_verts):
    return _make_sc_kernel(n_chunks, n_verts)


def kernel(verts, edges, verts_t, edges_t, N):
    e_count = edges.shape[0]
    assert e_count % _CHUNK_EDGES == 0, "edge count must be chunk-aligned"
    eidx = edges.astype(jnp.int32).reshape(-1, 128, 2).transpose(0, 2, 1)
    etidx = edges_t.astype(jnp.int32).reshape(-1, 128, 2).transpose(0, 2, 1)
    v_count = verts.shape[0]
    vp = -(-v_count // (_NS * _BLOCK_V)) * (_NS * _BLOCK_V)
    pad = ((0, vp - v_count), (0, 0))
    table = jnp.concatenate(
        [jnp.pad(verts, pad), jnp.pad(verts_t, pad),
         jnp.zeros((vp, 2), jnp.float32)], axis=1)
    tview = table.reshape(-1, 128, 8).transpose(0, 2, 1).reshape(-1)
    dummy = jnp.zeros((_CHUNK_WORDS, 8), jnp.float32)
    parts = _cached_sc_kernel(e_count // _CHUNK_EDGES, vp)(
        eidx, etidx, tview, dummy)
    return (jnp.sum(parts) / N).astype(jnp.float32)
